# baseline (speedup 1.0000x reference)
import functools

import jax
import jax.numpy as jnp
from jax import lax
from jax.experimental import pallas as pl
from jax.experimental.pallas import tpu as pltpu
from jax.experimental.pallas import tpu_sc as plsc

N_U = 10000
N_I = 10000
D = 128
LAYERS = 3
NNZ = 320000
B = 4096

NC = 2
NS = 16
EPT = NNZ // NS
K = 80
NB = 4000
NCB = NB // K
NBLK = EPT // NB
ROWS_PT = 632
ROWS_LAST = N_U - 15 * ROWS_PT
BPT = B // (NC * NS)

_MESH = plsc.VectorSubcoreMesh(
    core_axis_name="c", subcore_axis_name="s", num_cores=NC, num_subcores=NS
)


def _bcast_lane(vec16, j):
    idx = jnp.full((16, 1), j, jnp.int32)
    dn = lax.GatherDimensionNumbers(
        offset_dims=(), collapsed_slice_dims=(0,), start_index_map=(0,))
    return lax.gather(vec16, idx, dn, (1,),
                      mode=lax.GatherScatterMode.PROMISE_IN_BOUNDS)


@functools.partial(
    pl.kernel,
    out_type=jax.ShapeDtypeStruct((NC, N_U, D), jnp.float32),
    mesh=_MESH,
    scratch_types=[
        pltpu.VMEM_SHARED((N_U, D), jnp.float32),
        pltpu.VMEM((NB,), jnp.int32),
        pltpu.VMEM((NB,), jnp.int32),
        pltpu.VMEM((NB,), jnp.float32),
        pltpu.VMEM((K,), jnp.int32),
        pltpu.VMEM((K,), jnp.int32),
        pltpu.VMEM((K,), jnp.int32),
        pltpu.VMEM((K,), jnp.int32),
        pltpu.VMEM((K, D), jnp.float32),
        pltpu.VMEM((K, D), jnp.float32),
        pltpu.SemaphoreType.DMA,
        pltpu.SemaphoreType.DMA,
        pltpu.SemaphoreType.DMA,
        pltpu.SemaphoreType.DMA,
    ],
    compiler_params=pltpu.CompilerParams(needs_layout_passes=False),
)
def _spmm_kernel(rows_hbm, cols_hbm, vals_hbm, x_hbm, z_hbm,
                 acc, colb, rowb, valb, cv0, rv0, cv1, rv1, g0, g1,
                 sg0, sg1, ss0, ss1):
    cid = lax.axis_index("c")
    sid = lax.axis_index("s")

    zv = jnp.zeros((16,), jnp.float32)

    def _zrow(r, _):
        for dd in range(D // 16):
            g0[r, pl.ds(dd * 16, 16)] = zv
        return 0

    lax.fori_loop(0, K, _zrow, 0)
    r0 = sid * ROWS_PT

    @pl.when(sid < NS - 1)
    def _():
        for i in range(ROWS_PT // K):
            pltpu.sync_copy(g0, acc.at[pl.ds(r0 + i * K, K)])
        rem = ROWS_PT % K
        if rem:
            pltpu.sync_copy(g0.at[pl.ds(0, rem)],
                            acc.at[pl.ds(r0 + (ROWS_PT // K) * K, rem)])

    @pl.when(sid == NS - 1)
    def _():
        for i in range(ROWS_LAST // K):
            pltpu.sync_copy(g0, acc.at[pl.ds(r0 + i * K, K)])
        rem = ROWS_LAST % K
        if rem:
            pltpu.sync_copy(g0.at[pl.ds(0, rem)],
                            acc.at[pl.ds(r0 + (ROWS_LAST // K) * K, rem)])

    plsc.subcore_barrier()

    def _ldidx(c, cv, rv):
        for j in range(K // 16):
            sl_d = pl.ds(j * 16, 16)
            cv[sl_d] = colb[pl.ds(c * K + j * 16, 16)]
            rv[sl_d] = rowb[pl.ds(c * K + j * 16, 16)]

    def _scale(c, gv):
        def _grp(g, _):
            k0 = g * 16
            vv = valb[pl.ds(c * K + k0, 16)]
            for j in range(16):
                vb = _bcast_lane(vv, j)
                for dd in range(D // 16):
                    sl = pl.ds(dd * 16, 16)
                    gv[k0 + j, sl] = gv[k0 + j, sl] * vb
            return 0

        lax.fori_loop(0, K // 16, _grp, 0)

    def _block(b, _):
        ebase = cid * NNZ + sid * EPT + b * NB
        pltpu.sync_copy(cols_hbm.at[pl.ds(ebase, NB)], colb)
        pltpu.sync_copy(rows_hbm.at[pl.ds(ebase, NB)], rowb)
        pltpu.sync_copy(vals_hbm.at[pl.ds(ebase, NB)], valb)
        _ldidx(0, cv0, rv0)
        pltpu.async_copy(x_hbm.at[cv0], g0, sg0)
        _ldidx(1, cv1, rv1)
        pltpu.async_copy(x_hbm.at[cv1], g1, sg1)

        def _pair(p, _):
            pltpu.make_async_copy(x_hbm.at[cv0], g0, sg0).wait()
            _scale(2 * p, g0)
            pltpu.async_copy(g0, acc.at[rv0], ss0, add=True)

            pltpu.make_async_copy(x_hbm.at[cv1], g1, sg1).wait()
            _scale(2 * p + 1, g1)
            pltpu.async_copy(g1, acc.at[rv1], ss1, add=True)

            @pl.when(p < NCB // 2 - 1)
            def _():
                pltpu.make_async_copy(g0, acc.at[rv0], ss0).wait()
                _ldidx(2 * p + 2, cv0, rv0)
                pltpu.async_copy(x_hbm.at[cv0], g0, sg0)
                pltpu.make_async_copy(g1, acc.at[rv1], ss1).wait()
                _ldidx(2 * p + 3, cv1, rv1)
                pltpu.async_copy(x_hbm.at[cv1], g1, sg1)

            return 0

        lax.fori_loop(0, NCB // 2, _pair, 0)
        pltpu.make_async_copy(g0, acc.at[rv0], ss0).wait()
        pltpu.make_async_copy(g1, acc.at[rv1], ss1).wait()
        return 0

    lax.fori_loop(0, NBLK, _block, 0)
    plsc.subcore_barrier()

    @pl.when(sid < NS - 1)
    def _():
        pltpu.sync_copy(acc.at[pl.ds(r0, ROWS_PT)],
                        z_hbm.at[cid, pl.ds(r0, ROWS_PT)])

    @pl.when(sid == NS - 1)
    def _():
        pltpu.sync_copy(acc.at[pl.ds(r0, ROWS_LAST)],
                        z_hbm.at[cid, pl.ds(r0, ROWS_LAST)])


def _layer_tc_body(eu_ref, zp_ref, zn_ref, ei_ref,
                   ws_ref, w1_ref, w2_ref, w3_ref, w4_ref, wi_ref,
                   bu_ref, bi_ref, ou_ref, oi_ref):
    eu = eu_ref[...]
    zp = zp_ref[...]
    zn = zn_ref[...]
    ei = ei_ref[...]

    def mm(a, w):
        return lax.dot_general(a, w, (((1,), (1,)), ((), ())),
                               preferred_element_type=jnp.float32)

    mu = (mm(eu, ws_ref[...]) + mm(zp, w1_ref[...]) + mm(zp * eu, w2_ref[...])
          + mm(zn, w3_ref[...]) + mm(zn * eu, w4_ref[...]) + bu_ref[...])
    mi = mm(ei, wi_ref[...]) + bi_ref[...]
    ou_ref[...] = jnp.where(mu >= 0, mu, 0.1 * mu)
    oi_ref[...] = jnp.where(mi >= 0, mi, 0.1 * mi)


_ROWS_BLK = 1000
_N_BLK = N_U // _ROWS_BLK

_layer_tc = pl.pallas_call(
    _layer_tc_body,
    grid=(_N_BLK,),
    in_specs=[
        pl.BlockSpec((_ROWS_BLK, D), lambda i: (i, 0)),
        pl.BlockSpec((_ROWS_BLK, D), lambda i: (i, 0)),
        pl.BlockSpec((_ROWS_BLK, D), lambda i: (i, 0)),
        pl.BlockSpec((_ROWS_BLK, D), lambda i: (i, 0)),
    ] + [pl.BlockSpec((D, D), lambda i: (0, 0))] * 6
      + [pl.BlockSpec((1, D), lambda i: (0, 0))] * 2,
    out_specs=[
        pl.BlockSpec((_ROWS_BLK, D), lambda i: (i, 0)),
        pl.BlockSpec((_ROWS_BLK, D), lambda i: (i, 0)),
    ],
    out_shape=[
        jax.ShapeDtypeStruct((N_U, D), jnp.float32),
        jax.ShapeDtypeStruct((N_I, D), jnp.float32),
    ],
)


@functools.partial(
    pl.kernel,
    out_type=[
        jax.ShapeDtypeStruct((B, 16), jnp.float32),
        jax.ShapeDtypeStruct((B, 16), jnp.float32),
        jax.ShapeDtypeStruct((B, 16), jnp.float32),
    ],
    mesh=_MESH,
    scratch_types=[
        pltpu.VMEM((BPT,), jnp.int32),
        pltpu.VMEM((BPT,), jnp.int32),
        pltpu.VMEM((BPT, D), jnp.float32),
        pltpu.VMEM((BPT, D), jnp.float32),
        pltpu.VMEM((BPT, 16), jnp.float32),
        pltpu.VMEM((BPT, 16), jnp.float32),
        pltpu.VMEM((BPT, 16), jnp.float32),
        pltpu.SemaphoreType.DMA,
    ],
    compiler_params=pltpu.CompilerParams(needs_layout_passes=False),
)
def _score_kernel(eu_hbm, ei_hbm, uid_hbm, iid_hbm,
                  s_hbm, u2_hbm, i2_hbm,
                  uid_v, iid_v, u_v, i_v, s_v, u2_v, i2_v, sem):
    cid = lax.axis_index("c")
    sid = lax.axis_index("s")
    base = (sid * NC + cid) * BPT
    pltpu.sync_copy(uid_hbm.at[pl.ds(base, BPT)], uid_v)
    pltpu.sync_copy(iid_hbm.at[pl.ds(base, BPT)], iid_v)
    pltpu.async_copy(eu_hbm.at[uid_v], u_v, sem).wait()
    pltpu.async_copy(ei_hbm.at[iid_v], i_v, sem).wait()

    def _one(b, _):
        s = jnp.zeros((16,), jnp.float32)
        u2 = jnp.zeros((16,), jnp.float32)
        i2 = jnp.zeros((16,), jnp.float32)
        for dd in range(D // 16):
            sl = pl.ds(dd * 16, 16)
            u = u_v[b, sl]
            iv = i_v[b, sl]
            s = s + u * iv
            u2 = u2 + u * u
            i2 = i2 + iv * iv
        sl16 = pl.ds(0, 16)
        s_v[b, sl16] = jnp.full((16,), jnp.sum(s), jnp.float32)
        u2_v[b, sl16] = jnp.full((16,), jnp.sum(u2), jnp.float32)
        i2_v[b, sl16] = jnp.full((16,), jnp.sum(i2), jnp.float32)
        return 0

    lax.fori_loop(0, BPT, _one, 0)
    pltpu.sync_copy(s_v, s_hbm.at[pl.ds(base, BPT)])
    pltpu.sync_copy(u2_v, u2_hbm.at[pl.ds(base, BPT)])
    pltpu.sync_copy(i2_v, i2_hbm.at[pl.ds(base, BPT)])


def _loss_body(s_ref, u2_ref, i2_ref, y_ref, out_ref):
    s = s_ref[...]
    y = y_ref[...].astype(jnp.float32)
    bce = jnp.maximum(s, 0.0) - s * y + jnp.log1p(jnp.exp(-jnp.abs(s)))
    reg = jnp.mean(u2_ref[...]) + jnp.mean(i2_ref[...])
    out_ref[...] = jnp.full((1, 1), jnp.mean(bce) + 1e-6 * reg, jnp.float32)


_loss_tc = pl.pallas_call(
    _loss_body,
    out_shape=jax.ShapeDtypeStruct((1, 1), jnp.float32),
)


def kernel(uids, iids, labels, pos_rows, pos_cols, neg_rows, neg_cols,
           pos_vals, neg_vals, E_u_0, E_i_0,
           W_u_self_w, W_u_self_b, W_u_pos_1_w, W_u_pos_1_b,
           W_u_pos_2_w, W_u_pos_2_b, W_u_neg_3_w, W_u_neg_3_b,
           W_u_neg_4_w, W_u_neg_4_b, W_i_self_w, W_i_self_b):
    i32 = jnp.int32
    rows2 = jnp.concatenate([pos_rows, neg_rows]).astype(i32)
    cols2 = jnp.concatenate([pos_cols, neg_cols]).astype(i32)
    vals2 = jnp.concatenate([pos_vals, neg_vals]).astype(jnp.float32)

    E_u, E_i = E_u_0, E_i_0
    for l in range(LAYERS):
        z = _spmm_kernel(rows2, cols2, vals2, E_i)
        bu = (W_u_self_b[l] + W_u_pos_1_b[l] + W_u_pos_2_b[l]
              + W_u_neg_3_b[l] + W_u_neg_4_b[l])[None, :]
        E_u, E_i = _layer_tc(
            E_u, z[0], z[1], E_i,
            W_u_self_w[l], W_u_pos_1_w[l], W_u_pos_2_w[l],
            W_u_neg_3_w[l], W_u_neg_4_w[l], W_i_self_w[l],
            bu, W_i_self_b[l][None, :])

    s16, u216, i216 = _score_kernel(E_u, E_i, uids.astype(i32), iids.astype(i32))
    s = s16[:, 0]
    u2 = u216[:, 0]
    i2 = i216[:, 0]
    loss = _loss_tc(s.reshape(32, 128), u2.reshape(32, 128),
                    i2.reshape(32, 128), labels.reshape(32, 128))[0, 0]
    return (loss, s)


# speedup vs baseline: 1.1362x; 1.1362x over previous
import functools

import jax
import jax.numpy as jnp
from jax import lax
from jax.experimental import pallas as pl
from jax.experimental.pallas import tpu as pltpu
from jax.experimental.pallas import tpu_sc as plsc

N_U = 10000
N_I = 10000
D = 128
LAYERS = 3
NNZ = 320000
B = 4096

NC = 2
NS = 16
EPT = NNZ // NS
K = 80
NB = 4000
NCB = NB // K
NBLK = EPT // NB
ROWS_PT = 632
ROWS_LAST = N_U - 15 * ROWS_PT
BPT = B // (NC * NS)

_MESH = plsc.VectorSubcoreMesh(
    core_axis_name="c", subcore_axis_name="s", num_cores=NC, num_subcores=NS
)


def _bcast_lane(vec16, j):
    idx = jnp.full((16, 1), j, jnp.int32)
    dn = lax.GatherDimensionNumbers(
        offset_dims=(), collapsed_slice_dims=(0,), start_index_map=(0,))
    return lax.gather(vec16, idx, dn, (1,),
                      mode=lax.GatherScatterMode.PROMISE_IN_BOUNDS)


@functools.partial(
    pl.kernel,
    out_type=jax.ShapeDtypeStruct((NC, N_U, D), jnp.float32),
    mesh=_MESH,
    scratch_types=[
        pltpu.VMEM_SHARED((N_U, D), jnp.float32),
        pltpu.VMEM((NB,), jnp.int32),
        pltpu.VMEM((NB,), jnp.int32),
        pltpu.VMEM((NB,), jnp.float32),
        pltpu.VMEM((K,), jnp.int32),
        pltpu.VMEM((K,), jnp.int32),
        pltpu.VMEM((K,), jnp.int32),
        pltpu.VMEM((K,), jnp.int32),
        pltpu.VMEM((K, D), jnp.float32),
        pltpu.VMEM((K, D), jnp.float32),
        pltpu.SemaphoreType.DMA,
        pltpu.SemaphoreType.DMA,
    ],
    compiler_params=pltpu.CompilerParams(needs_layout_passes=False),
)
def _spmm_kernel(rows_hbm, cols_hbm, vals_hbm, x_hbm, z_hbm,
                 acc, colb, rowb, valb, cv0, rv0, cv1, rv1, g0, g1,
                 sg0, sg1):
    cid = lax.axis_index("c")
    sid = lax.axis_index("s")

    zv = jnp.zeros((16,), jnp.float32)

    def _zrow(r, _):
        for dd in range(D // 16):
            g0[r, pl.ds(dd * 16, 16)] = zv
        return 0

    lax.fori_loop(0, K, _zrow, 0)
    r0 = sid * ROWS_PT

    @pl.when(sid < NS - 1)
    def _():
        for i in range(ROWS_PT // K):
            pltpu.sync_copy(g0, acc.at[pl.ds(r0 + i * K, K)])
        rem = ROWS_PT % K
        if rem:
            pltpu.sync_copy(g0.at[pl.ds(0, rem)],
                            acc.at[pl.ds(r0 + (ROWS_PT // K) * K, rem)])

    @pl.when(sid == NS - 1)
    def _():
        for i in range(ROWS_LAST // K):
            pltpu.sync_copy(g0, acc.at[pl.ds(r0 + i * K, K)])
        rem = ROWS_LAST % K
        if rem:
            pltpu.sync_copy(g0.at[pl.ds(0, rem)],
                            acc.at[pl.ds(r0 + (ROWS_LAST // K) * K, rem)])

    plsc.subcore_barrier()

    def _ldidx(c, cv, rv):
        for j in range(K // 16):
            sl_d = pl.ds(j * 16, 16)
            cv[sl_d] = colb[pl.ds(c * K + j * 16, 16)]
            rv[sl_d] = rowb[pl.ds(c * K + j * 16, 16)]

    def _scale(c, gv):
        def _grp(g, _):
            k0 = g * 16
            vv = valb[pl.ds(c * K + k0, 16)]
            for j in range(16):
                vb = _bcast_lane(vv, j)
                for dd in range(D // 16):
                    sl = pl.ds(dd * 16, 16)
                    gv[k0 + j, sl] = gv[k0 + j, sl] * vb
            return 0

        lax.fori_loop(0, K // 16, _grp, 0)

    def _block(b, _):
        ebase = cid * NNZ + sid * EPT + b * NB
        pltpu.sync_copy(cols_hbm.at[pl.ds(ebase, NB)], colb)
        pltpu.sync_copy(rows_hbm.at[pl.ds(ebase, NB)], rowb)
        pltpu.sync_copy(vals_hbm.at[pl.ds(ebase, NB)], valb)
        _ldidx(0, cv0, rv0)
        pltpu.async_copy(x_hbm.at[cv0], g0, sg0)
        _ldidx(1, cv1, rv1)
        pltpu.async_copy(x_hbm.at[cv1], g1, sg1)

        def _pair(p, _):
            pltpu.make_async_copy(x_hbm.at[cv0], g0, sg0).wait()
            _scale(2 * p, g0)
            pltpu.sync_copy(g0, acc.at[rv0], add=True)

            @pl.when(p < NCB // 2 - 1)
            def _():
                _ldidx(2 * p + 2, cv0, rv0)
                pltpu.async_copy(x_hbm.at[cv0], g0, sg0)

            pltpu.make_async_copy(x_hbm.at[cv1], g1, sg1).wait()
            _scale(2 * p + 1, g1)
            pltpu.sync_copy(g1, acc.at[rv1], add=True)

            @pl.when(p < NCB // 2 - 1)
            def _():
                _ldidx(2 * p + 3, cv1, rv1)
                pltpu.async_copy(x_hbm.at[cv1], g1, sg1)

            return 0

        lax.fori_loop(0, NCB // 2, _pair, 0)
        return 0

    lax.fori_loop(0, NBLK, _block, 0)
    plsc.subcore_barrier()

    @pl.when(sid < NS - 1)
    def _():
        pltpu.sync_copy(acc.at[pl.ds(r0, ROWS_PT)],
                        z_hbm.at[cid, pl.ds(r0, ROWS_PT)])

    @pl.when(sid == NS - 1)
    def _():
        pltpu.sync_copy(acc.at[pl.ds(r0, ROWS_LAST)],
                        z_hbm.at[cid, pl.ds(r0, ROWS_LAST)])


_ROWS_BLK = 1000
_N_BLK = N_U // _ROWS_BLK


def _ei_chain_body(ei_ref, w_ref, b_ref, o1_ref, o2_ref, o3_ref):
    x = ei_ref[...]
    outs = (o1_ref, o2_ref, o3_ref)
    for l in range(LAYERS):
        m = lax.dot_general(x, w_ref[l], (((1,), (1,)), ((), ())),
                            preferred_element_type=jnp.float32) + b_ref[l]
        x = jnp.where(m >= 0, m, 0.1 * m)
        outs[l][...] = x


_ei_chain = pl.pallas_call(
    _ei_chain_body,
    grid=(_N_BLK,),
    in_specs=[
        pl.BlockSpec((_ROWS_BLK, D), lambda i: (i, 0)),
        pl.BlockSpec((LAYERS, D, D), lambda i: (0, 0, 0)),
        pl.BlockSpec((LAYERS, 1, D), lambda i: (0, 0, 0)),
    ],
    out_specs=[pl.BlockSpec((_ROWS_BLK, D), lambda i: (i, 0))] * 3,
    out_shape=[jax.ShapeDtypeStruct((N_I, D), jnp.float32)] * 3,
)


def _layer_tc_body(eu_ref, zp_ref, zn_ref,
                   ws_ref, w1_ref, w2_ref, w3_ref, w4_ref,
                   bu_ref, ou_ref):
    eu = eu_ref[...]
    zp = zp_ref[...]
    zn = zn_ref[...]

    def mm(a, w):
        return lax.dot_general(a, w, (((1,), (1,)), ((), ())),
                               preferred_element_type=jnp.float32)

    mu = (mm(eu, ws_ref[...]) + mm(zp, w1_ref[...]) + mm(zp * eu, w2_ref[...])
          + mm(zn, w3_ref[...]) + mm(zn * eu, w4_ref[...]) + bu_ref[...])
    ou_ref[...] = jnp.where(mu >= 0, mu, 0.1 * mu)


_layer_tc = pl.pallas_call(
    _layer_tc_body,
    grid=(_N_BLK,),
    in_specs=[
        pl.BlockSpec((_ROWS_BLK, D), lambda i: (i, 0)),
        pl.BlockSpec((_ROWS_BLK, D), lambda i: (i, 0)),
        pl.BlockSpec((_ROWS_BLK, D), lambda i: (i, 0)),
    ] + [pl.BlockSpec((D, D), lambda i: (0, 0))] * 5
      + [pl.BlockSpec((1, D), lambda i: (0, 0))],
    out_specs=pl.BlockSpec((_ROWS_BLK, D), lambda i: (i, 0)),
    out_shape=jax.ShapeDtypeStruct((N_U, D), jnp.float32),
)


@functools.partial(
    pl.kernel,
    out_type=[
        jax.ShapeDtypeStruct((B, 16), jnp.float32),
        jax.ShapeDtypeStruct((B, 16), jnp.float32),
        jax.ShapeDtypeStruct((B, 16), jnp.float32),
    ],
    mesh=_MESH,
    scratch_types=[
        pltpu.VMEM((BPT,), jnp.int32),
        pltpu.VMEM((BPT,), jnp.int32),
        pltpu.VMEM((BPT, D), jnp.float32),
        pltpu.VMEM((BPT, D), jnp.float32),
        pltpu.VMEM((BPT, 16), jnp.float32),
        pltpu.VMEM((BPT, 16), jnp.float32),
        pltpu.VMEM((BPT, 16), jnp.float32),
        pltpu.SemaphoreType.DMA,
    ],
    compiler_params=pltpu.CompilerParams(needs_layout_passes=False),
)
def _score_kernel(eu_hbm, ei_hbm, uid_hbm, iid_hbm,
                  s_hbm, u2_hbm, i2_hbm,
                  uid_v, iid_v, u_v, i_v, s_v, u2_v, i2_v, sem):
    cid = lax.axis_index("c")
    sid = lax.axis_index("s")
    base = (sid * NC + cid) * BPT
    pltpu.sync_copy(uid_hbm.at[pl.ds(base, BPT)], uid_v)
    pltpu.sync_copy(iid_hbm.at[pl.ds(base, BPT)], iid_v)
    pltpu.async_copy(eu_hbm.at[uid_v], u_v, sem).wait()
    pltpu.async_copy(ei_hbm.at[iid_v], i_v, sem).wait()

    def _one(b, _):
        s = jnp.zeros((16,), jnp.float32)
        u2 = jnp.zeros((16,), jnp.float32)
        i2 = jnp.zeros((16,), jnp.float32)
        for dd in range(D // 16):
            sl = pl.ds(dd * 16, 16)
            u = u_v[b, sl]
            iv = i_v[b, sl]
            s = s + u * iv
            u2 = u2 + u * u
            i2 = i2 + iv * iv
        sl16 = pl.ds(0, 16)
        s_v[b, sl16] = jnp.full((16,), jnp.sum(s), jnp.float32)
        u2_v[b, sl16] = jnp.full((16,), jnp.sum(u2), jnp.float32)
        i2_v[b, sl16] = jnp.full((16,), jnp.sum(i2), jnp.float32)
        return 0

    lax.fori_loop(0, BPT, _one, 0)
    pltpu.sync_copy(s_v, s_hbm.at[pl.ds(base, BPT)])
    pltpu.sync_copy(u2_v, u2_hbm.at[pl.ds(base, BPT)])
    pltpu.sync_copy(i2_v, i2_hbm.at[pl.ds(base, BPT)])


def _loss_body(s_ref, u2_ref, i2_ref, y_ref, out_ref):
    s = s_ref[...]
    y = y_ref[...].astype(jnp.float32)
    bce = jnp.maximum(s, 0.0) - s * y + jnp.log1p(jnp.exp(-jnp.abs(s)))
    reg = jnp.mean(u2_ref[...]) + jnp.mean(i2_ref[...])
    out_ref[...] = jnp.full((1, 1), jnp.mean(bce) + 1e-6 * reg, jnp.float32)


_loss_tc = pl.pallas_call(
    _loss_body,
    out_shape=jax.ShapeDtypeStruct((1, 1), jnp.float32),
)


def kernel(uids, iids, labels, pos_rows, pos_cols, neg_rows, neg_cols,
           pos_vals, neg_vals, E_u_0, E_i_0,
           W_u_self_w, W_u_self_b, W_u_pos_1_w, W_u_pos_1_b,
           W_u_pos_2_w, W_u_pos_2_b, W_u_neg_3_w, W_u_neg_3_b,
           W_u_neg_4_w, W_u_neg_4_b, W_i_self_w, W_i_self_b):
    i32 = jnp.int32
    rows2 = jnp.concatenate([pos_rows, neg_rows]).astype(i32)
    cols2 = jnp.concatenate([pos_cols, neg_cols]).astype(i32)
    vals2 = jnp.concatenate([pos_vals, neg_vals]).astype(jnp.float32)

    eis = _ei_chain(E_i_0, W_i_self_w, W_i_self_b[:, None, :])
    ei_in = (E_i_0, eis[0], eis[1])

    E_u = E_u_0
    for l in range(LAYERS):
        z = _spmm_kernel(rows2, cols2, vals2, ei_in[l])
        bu = (W_u_self_b[l] + W_u_pos_1_b[l] + W_u_pos_2_b[l]
              + W_u_neg_3_b[l] + W_u_neg_4_b[l])[None, :]
        E_u = _layer_tc(
            E_u, z[0], z[1],
            W_u_self_w[l], W_u_pos_1_w[l], W_u_pos_2_w[l],
            W_u_neg_3_w[l], W_u_neg_4_w[l], bu)

    s16, u216, i216 = _score_kernel(E_u, eis[2], uids.astype(i32), iids.astype(i32))
    s = s16[:, 0]
    u2 = u216[:, 0]
    i2 = i216[:, 0]
    loss = _loss_tc(s.reshape(32, 128), u2.reshape(32, 128),
                    i2.reshape(32, 128), labels.reshape(32, 128))[0, 0]
    return (loss, s)


# speedup vs baseline: 1.2441x; 1.0950x over previous
import functools

import jax
import jax.numpy as jnp
from jax import lax
from jax.experimental import pallas as pl
from jax.experimental.pallas import tpu as pltpu
from jax.experimental.pallas import tpu_sc as plsc

N_U = 10000
N_I = 10000
D = 128
LAYERS = 3
NNZ = 320000
B = 4096

NC = 2
NS = 16
EPT = NNZ // NS
K = 80
NB = 4000
NCB = NB // K
NBLK = EPT // NB
ROWS_PT = 632
ROWS_LAST = N_U - 15 * ROWS_PT
BPT = B // (NC * NS)

_MESH = plsc.VectorSubcoreMesh(
    core_axis_name="c", subcore_axis_name="s", num_cores=NC, num_subcores=NS
)


def _bcast_lane(vec16, j):
    idx = jnp.full((16, 1), j, jnp.int32)
    dn = lax.GatherDimensionNumbers(
        offset_dims=(), collapsed_slice_dims=(0,), start_index_map=(0,))
    return lax.gather(vec16, idx, dn, (1,),
                      mode=lax.GatherScatterMode.PROMISE_IN_BOUNDS)


@functools.partial(
    pl.kernel,
    out_type=jax.ShapeDtypeStruct((NC, N_U, D), jnp.float32),
    mesh=_MESH,
    scratch_types=[
        pltpu.VMEM_SHARED((N_U, D), jnp.float32),
        pltpu.VMEM((NB,), jnp.int32),
        pltpu.VMEM((NB,), jnp.int32),
        pltpu.VMEM((NB,), jnp.float32),
        pltpu.VMEM((K,), jnp.int32),
        pltpu.VMEM((K,), jnp.int32),
        pltpu.VMEM((K,), jnp.int32),
        pltpu.VMEM((K,), jnp.int32),
        pltpu.VMEM((K,), jnp.int32),
        pltpu.VMEM((K,), jnp.int32),
        pltpu.VMEM((K, D), jnp.float32),
        pltpu.VMEM((K, D), jnp.float32),
        pltpu.VMEM((K, D), jnp.float32),
        pltpu.SemaphoreType.DMA,
        pltpu.SemaphoreType.DMA,
        pltpu.SemaphoreType.DMA,
    ],
    compiler_params=pltpu.CompilerParams(needs_layout_passes=False),
)
def _spmm_kernel(rows_hbm, cols_hbm, vals_hbm, x_hbm, z_hbm,
                 acc, colb, rowb, valb, cv0, rv0, cv1, rv1, cv2, rv2,
                 g0, g1, g2, sg0, sg1, sg2):
    cid = lax.axis_index("c")
    sid = lax.axis_index("s")

    zv = jnp.zeros((16,), jnp.float32)

    def _zrow(r, _):
        for dd in range(D // 16):
            g0[r, pl.ds(dd * 16, 16)] = zv
        return 0

    lax.fori_loop(0, K, _zrow, 0)
    r0 = sid * ROWS_PT

    @pl.when(sid < NS - 1)
    def _():
        for i in range(ROWS_PT // K):
            pltpu.sync_copy(g0, acc.at[pl.ds(r0 + i * K, K)])
        rem = ROWS_PT % K
        if rem:
            pltpu.sync_copy(g0.at[pl.ds(0, rem)],
                            acc.at[pl.ds(r0 + (ROWS_PT // K) * K, rem)])

    @pl.when(sid == NS - 1)
    def _():
        for i in range(ROWS_LAST // K):
            pltpu.sync_copy(g0, acc.at[pl.ds(r0 + i * K, K)])
        rem = ROWS_LAST % K
        if rem:
            pltpu.sync_copy(g0.at[pl.ds(0, rem)],
                            acc.at[pl.ds(r0 + (ROWS_LAST // K) * K, rem)])

    plsc.subcore_barrier()

    def _ldidx(c, cv, rv):
        for j in range(K // 16):
            sl_d = pl.ds(j * 16, 16)
            cv[sl_d] = colb[pl.ds(c * K + j * 16, 16)]
            rv[sl_d] = rowb[pl.ds(c * K + j * 16, 16)]

    def _scale(c, gv):
        def _grp(g, _):
            k0 = g * 16
            vv = valb[pl.ds(c * K + k0, 16)]
            for j in range(16):
                vb = _bcast_lane(vv, j)
                for dd in range(D // 16):
                    sl = pl.ds(dd * 16, 16)
                    gv[k0 + j, sl] = gv[k0 + j, sl] * vb
            return 0

        lax.fori_loop(0, K // 16, _grp, 0)

    def _block(b, _):
        ebase = cid * NNZ + sid * EPT + b * NB
        pltpu.sync_copy(cols_hbm.at[pl.ds(ebase, NB)], colb)
        pltpu.sync_copy(rows_hbm.at[pl.ds(ebase, NB)], rowb)
        pltpu.sync_copy(vals_hbm.at[pl.ds(ebase, NB)], valb)
        _ldidx(0, cv0, rv0)
        pltpu.async_copy(x_hbm.at[cv0], g0, sg0)
        _ldidx(1, cv1, rv1)
        pltpu.async_copy(x_hbm.at[cv1], g1, sg1)
        _ldidx(2, cv2, rv2)
        pltpu.async_copy(x_hbm.at[cv2], g2, sg2)

        def _one(c, cv, rv, gv, sg):
            pltpu.make_async_copy(x_hbm.at[cv], gv, sg).wait()
            _scale(c, gv)
            pltpu.sync_copy(gv, acc.at[rv], add=True)

            @pl.when(c + 3 < NCB)
            def _():
                _ldidx(c + 3, cv, rv)
                pltpu.async_copy(x_hbm.at[cv], gv, sg)

        def _triple(p, _):
            _one(3 * p, cv0, rv0, g0, sg0)
            _one(3 * p + 1, cv1, rv1, g1, sg1)
            _one(3 * p + 2, cv2, rv2, g2, sg2)
            return 0

        lax.fori_loop(0, NCB // 3, _triple, 0)
        for r in range(NCB % 3):
            c = (NCB // 3) * 3 + r
            cv, rv, gv, sg = ((cv0, rv0, g0, sg0), (cv1, rv1, g1, sg1),
                              (cv2, rv2, g2, sg2))[r]
            pltpu.make_async_copy(x_hbm.at[cv], gv, sg).wait()
            _scale(c, gv)
            pltpu.sync_copy(gv, acc.at[rv], add=True)
        return 0

    lax.fori_loop(0, NBLK, _block, 0)
    plsc.subcore_barrier()

    @pl.when(sid < NS - 1)
    def _():
        pltpu.sync_copy(acc.at[pl.ds(r0, ROWS_PT)],
                        z_hbm.at[cid, pl.ds(r0, ROWS_PT)])

    @pl.when(sid == NS - 1)
    def _():
        pltpu.sync_copy(acc.at[pl.ds(r0, ROWS_LAST)],
                        z_hbm.at[cid, pl.ds(r0, ROWS_LAST)])


_ROWS_BLK = 1000
_N_BLK = N_U // _ROWS_BLK


def _ei_chain_body(ei_ref, w_ref, b_ref, o1_ref, o2_ref, o3_ref):
    x = ei_ref[...]
    outs = (o1_ref, o2_ref, o3_ref)
    for l in range(LAYERS):
        m = lax.dot_general(x, w_ref[l], (((1,), (1,)), ((), ())),
                            preferred_element_type=jnp.float32) + b_ref[l]
        x = jnp.where(m >= 0, m, 0.1 * m)
        outs[l][...] = x


_ei_chain = pl.pallas_call(
    _ei_chain_body,
    grid=(_N_BLK,),
    in_specs=[
        pl.BlockSpec((_ROWS_BLK, D), lambda i: (i, 0)),
        pl.BlockSpec((LAYERS, D, D), lambda i: (0, 0, 0)),
        pl.BlockSpec((LAYERS, 1, D), lambda i: (0, 0, 0)),
    ],
    out_specs=[pl.BlockSpec((_ROWS_BLK, D), lambda i: (i, 0))] * 3,
    out_shape=[jax.ShapeDtypeStruct((N_I, D), jnp.float32)] * 3,
)


def _layer_tc_body(eu_ref, zp_ref, zn_ref,
                   ws_ref, w1_ref, w2_ref, w3_ref, w4_ref,
                   bu_ref, ou_ref):
    eu = eu_ref[...]
    zp = zp_ref[...]
    zn = zn_ref[...]

    def mm(a, w):
        return lax.dot_general(a, w, (((1,), (1,)), ((), ())),
                               preferred_element_type=jnp.float32)

    mu = (mm(eu, ws_ref[...]) + mm(zp, w1_ref[...]) + mm(zp * eu, w2_ref[...])
          + mm(zn, w3_ref[...]) + mm(zn * eu, w4_ref[...]) + bu_ref[...])
    ou_ref[...] = jnp.where(mu >= 0, mu, 0.1 * mu)


_layer_tc = pl.pallas_call(
    _layer_tc_body,
    grid=(_N_BLK,),
    in_specs=[
        pl.BlockSpec((_ROWS_BLK, D), lambda i: (i, 0)),
        pl.BlockSpec((_ROWS_BLK, D), lambda i: (i, 0)),
        pl.BlockSpec((_ROWS_BLK, D), lambda i: (i, 0)),
    ] + [pl.BlockSpec((D, D), lambda i: (0, 0))] * 5
      + [pl.BlockSpec((1, D), lambda i: (0, 0))],
    out_specs=pl.BlockSpec((_ROWS_BLK, D), lambda i: (i, 0)),
    out_shape=jax.ShapeDtypeStruct((N_U, D), jnp.float32),
)


@functools.partial(
    pl.kernel,
    out_type=[
        jax.ShapeDtypeStruct((B, 16), jnp.float32),
        jax.ShapeDtypeStruct((B, 16), jnp.float32),
        jax.ShapeDtypeStruct((B, 16), jnp.float32),
    ],
    mesh=_MESH,
    scratch_types=[
        pltpu.VMEM((BPT,), jnp.int32),
        pltpu.VMEM((BPT,), jnp.int32),
        pltpu.VMEM((BPT, D), jnp.float32),
        pltpu.VMEM((BPT, D), jnp.float32),
        pltpu.VMEM((BPT, 16), jnp.float32),
        pltpu.VMEM((BPT, 16), jnp.float32),
        pltpu.VMEM((BPT, 16), jnp.float32),
        pltpu.SemaphoreType.DMA,
    ],
    compiler_params=pltpu.CompilerParams(needs_layout_passes=False),
)
def _score_kernel(eu_hbm, ei_hbm, uid_hbm, iid_hbm,
                  s_hbm, u2_hbm, i2_hbm,
                  uid_v, iid_v, u_v, i_v, s_v, u2_v, i2_v, sem):
    cid = lax.axis_index("c")
    sid = lax.axis_index("s")
    base = (sid * NC + cid) * BPT
    pltpu.sync_copy(uid_hbm.at[pl.ds(base, BPT)], uid_v)
    pltpu.sync_copy(iid_hbm.at[pl.ds(base, BPT)], iid_v)
    pltpu.async_copy(eu_hbm.at[uid_v], u_v, sem).wait()
    pltpu.async_copy(ei_hbm.at[iid_v], i_v, sem).wait()

    def _one(b, _):
        s = jnp.zeros((16,), jnp.float32)
        u2 = jnp.zeros((16,), jnp.float32)
        i2 = jnp.zeros((16,), jnp.float32)
        for dd in range(D // 16):
            sl = pl.ds(dd * 16, 16)
            u = u_v[b, sl]
            iv = i_v[b, sl]
            s = s + u * iv
            u2 = u2 + u * u
            i2 = i2 + iv * iv
        sl16 = pl.ds(0, 16)
        s_v[b, sl16] = jnp.full((16,), jnp.sum(s), jnp.float32)
        u2_v[b, sl16] = jnp.full((16,), jnp.sum(u2), jnp.float32)
        i2_v[b, sl16] = jnp.full((16,), jnp.sum(i2), jnp.float32)
        return 0

    lax.fori_loop(0, BPT, _one, 0)
    pltpu.sync_copy(s_v, s_hbm.at[pl.ds(base, BPT)])
    pltpu.sync_copy(u2_v, u2_hbm.at[pl.ds(base, BPT)])
    pltpu.sync_copy(i2_v, i2_hbm.at[pl.ds(base, BPT)])


def _loss_body(s_ref, u2_ref, i2_ref, y_ref, out_ref):
    s = s_ref[...]
    y = y_ref[...].astype(jnp.float32)
    bce = jnp.maximum(s, 0.0) - s * y + jnp.log1p(jnp.exp(-jnp.abs(s)))
    reg = jnp.mean(u2_ref[...]) + jnp.mean(i2_ref[...])
    out_ref[...] = jnp.full((1, 1), jnp.mean(bce) + 1e-6 * reg, jnp.float32)


_loss_tc = pl.pallas_call(
    _loss_body,
    out_shape=jax.ShapeDtypeStruct((1, 1), jnp.float32),
)


def kernel(uids, iids, labels, pos_rows, pos_cols, neg_rows, neg_cols,
           pos_vals, neg_vals, E_u_0, E_i_0,
           W_u_self_w, W_u_self_b, W_u_pos_1_w, W_u_pos_1_b,
           W_u_pos_2_w, W_u_pos_2_b, W_u_neg_3_w, W_u_neg_3_b,
           W_u_neg_4_w, W_u_neg_4_b, W_i_self_w, W_i_self_b):
    i32 = jnp.int32
    rows2 = jnp.concatenate([pos_rows, neg_rows]).astype(i32)
    cols2 = jnp.concatenate([pos_cols, neg_cols]).astype(i32)
    vals2 = jnp.concatenate([pos_vals, neg_vals]).astype(jnp.float32)

    eis = _ei_chain(E_i_0, W_i_self_w, W_i_self_b[:, None, :])
    ei_in = (E_i_0, eis[0], eis[1])

    E_u = E_u_0
    for l in range(LAYERS):
        z = _spmm_kernel(rows2, cols2, vals2, ei_in[l])
        bu = (W_u_self_b[l] + W_u_pos_1_b[l] + W_u_pos_2_b[l]
              + W_u_neg_3_b[l] + W_u_neg_4_b[l])[None, :]
        E_u = _layer_tc(
            E_u, z[0], z[1],
            W_u_self_w[l], W_u_pos_1_w[l], W_u_pos_2_w[l],
            W_u_neg_3_w[l], W_u_neg_4_w[l], bu)

    s16, u216, i216 = _score_kernel(E_u, eis[2], uids.astype(i32), iids.astype(i32))
    s = s16[:, 0]
    u2 = u216[:, 0]
    i2 = i216[:, 0]
    loss = _loss_tc(s.reshape(32, 128), u2.reshape(32, 128),
                    i2.reshape(32, 128), labels.reshape(32, 128))[0, 0]
    return (loss, s)
